# Initial kernel scaffold; baseline (speedup 1.0000x reference)
import jax
import jax.numpy as jnp
from jax.experimental import pallas as pl


def kernel(x, edge_index, W1, b1, W2, b2):
    raise NotImplementedError("write your pallas kernel here")


# speedup vs baseline: 22.2909x; 22.2909x over previous
import functools

import jax
import jax.numpy as jnp
from jax import lax
from jax.experimental import pallas as pl
from jax.experimental.pallas import tpu as pltpu
from jax.experimental.pallas import tpu_sc as plsc

N = 10000
NPAD = 10240
D = 128
E = 320000

NC = 2
NS = 16
NW = NC * NS
EPW = E // NW
CHUNK = 80
NCHUNK = EPW // CHUNK
ROWS_PT = NPAD // NS

_MESH = plsc.VectorSubcoreMesh(
    core_axis_name="c", subcore_axis_name="s", num_cores=NC, num_subcores=NS)


def _deg_body(dst_hbm, ones_hbm, zeros_hbm, out_hbm,
              didx0, didx1, ones_v, semi0, semi1, acc):
  c = lax.axis_index("c")
  s = lax.axis_index("s")
  w = c * NS + s
  base = w * EPW

  def idx_load(j, didx, semi):
    pltpu.async_copy(dst_hbm.at[pl.ds(base + j * CHUNK, CHUNK)], didx, semi)

  def idx_wait(didx, semi):
    pltpu.make_async_copy(dst_hbm.at[pl.ds(base, CHUNK)], didx, semi).wait()

  idx_load(0, didx0, semi0)
  idx_load(1, didx1, semi1)
  pltpu.sync_copy(ones_hbm, ones_v)
  pltpu.sync_copy(zeros_hbm.at[pl.ds(s * ROWS_PT, ROWS_PT)],
                  acc.at[pl.ds(s * ROWS_PT, ROWS_PT)])
  plsc.subcore_barrier()

  def step(j, didx_a, semi_a, didx_b, semi_b):
    idx_wait(didx_a, semi_a)
    pltpu.sync_copy(ones_v, acc.at[didx_a], add=True)

    @pl.when(j + 2 < NCHUNK)
    def _():
      idx_load(j + 2, didx_a, semi_a)

  def body(jj, carry):
    j0 = jj * 2
    step(j0, didx0, semi0, didx1, semi1)

    @pl.when(j0 + 1 < NCHUNK)
    def _():
      step(j0 + 1, didx1, semi1, didx0, semi0)

    return carry

  lax.fori_loop(0, (NCHUNK + 1) // 2, body, 0)
  plsc.subcore_barrier()
  pltpu.sync_copy(acc.at[pl.ds(s * ROWS_PT, ROWS_PT)],
                  out_hbm.at[c, pl.ds(s * ROWS_PT, ROWS_PT)])


_deg_call = pl.kernel(
    _deg_body,
    out_type=jax.ShapeDtypeStruct((NC, NPAD, D), jnp.float32),
    mesh=_MESH,
    scratch_types=[
        pltpu.VMEM((CHUNK,), jnp.int32),
        pltpu.VMEM((CHUNK,), jnp.int32),
        pltpu.VMEM((CHUNK, D), jnp.float32),
        pltpu.SemaphoreType.DMA,
        pltpu.SemaphoreType.DMA,
        pltpu.VMEM_SHARED((NPAD, D), jnp.float32),
    ],
)


def _agg_body(g_hbm, src_hbm, dst_hbm, zeros_hbm, out_hbm,
              sidx0, sidx1, didx0, didx1, rows0, rows1,
              semi0, semi1, semg0, semg1, acc):
  c = lax.axis_index("c")
  s = lax.axis_index("s")
  w = c * NS + s
  base = w * EPW

  def idx_load(j, sidx, didx, semi):
    pltpu.async_copy(src_hbm.at[pl.ds(base + j * CHUNK, CHUNK)], sidx, semi)
    pltpu.async_copy(dst_hbm.at[pl.ds(base + j * CHUNK, CHUNK)], didx, semi)

  def idx_wait(sidx, didx, semi):
    pltpu.make_async_copy(src_hbm.at[pl.ds(base, CHUNK)], sidx, semi).wait()
    pltpu.make_async_copy(dst_hbm.at[pl.ds(base, CHUNK)], didx, semi).wait()

  idx_load(0, sidx0, didx0, semi0)
  pltpu.sync_copy(zeros_hbm.at[pl.ds(s * ROWS_PT, ROWS_PT)],
                  acc.at[pl.ds(s * ROWS_PT, ROWS_PT)])
  idx_wait(sidx0, didx0, semi0)
  plsc.subcore_barrier()
  pltpu.async_copy(g_hbm.at[sidx0], rows0, semg0)
  idx_load(1, sidx1, didx1, semi1)

  def step(j, sidx_a, didx_a, rows_a, semi_a, semg_a,
           sidx_b, didx_b, rows_b, semi_b, semg_b):
    pltpu.make_async_copy(g_hbm.at[sidx_a], rows_a, semg_a).wait()

    @pl.when(j + 1 < NCHUNK)
    def _():
      idx_wait(sidx_b, didx_b, semi_b)
      pltpu.async_copy(g_hbm.at[sidx_b], rows_b, semg_b)

    pltpu.sync_copy(rows_a, acc.at[didx_a], add=True)

    @pl.when(j + 2 < NCHUNK)
    def _():
      idx_load(j + 2, sidx_a, didx_a, semi_a)

  def body(jj, carry):
    j0 = jj * 2
    step(j0, sidx0, didx0, rows0, semi0, semg0,
         sidx1, didx1, rows1, semi1, semg1)

    @pl.when(j0 + 1 < NCHUNK)
    def _():
      step(j0 + 1, sidx1, didx1, rows1, semi1, semg1,
           sidx0, didx0, rows0, semi0, semg0)

    return carry

  lax.fori_loop(0, (NCHUNK + 1) // 2, body, 0)
  plsc.subcore_barrier()
  pltpu.sync_copy(acc.at[pl.ds(s * ROWS_PT, ROWS_PT)],
                  out_hbm.at[c, pl.ds(s * ROWS_PT, ROWS_PT)])


_agg_call = pl.kernel(
    _agg_body,
    out_type=jax.ShapeDtypeStruct((NC, NPAD, D), jnp.float32),
    mesh=_MESH,
    scratch_types=[
        pltpu.VMEM((CHUNK,), jnp.int32),
        pltpu.VMEM((CHUNK,), jnp.int32),
        pltpu.VMEM((CHUNK,), jnp.int32),
        pltpu.VMEM((CHUNK,), jnp.int32),
        pltpu.VMEM((CHUNK, D), jnp.float32),
        pltpu.VMEM((CHUNK, D), jnp.float32),
        pltpu.SemaphoreType.DMA,
        pltpu.SemaphoreType.DMA,
        pltpu.SemaphoreType.DMA,
        pltpu.SemaphoreType.DMA,
        pltpu.VMEM_SHARED((NPAD, D), jnp.float32),
    ],
)


BR = 1000


def _dinv(degp_ref):
  deg = degp_ref[0, :, 0:1] + degp_ref[1, :, 0:1] + 1.0
  return lax.rsqrt(deg)


def _tc1_body(x_ref, w_ref, degp_ref, g_ref):
  dinv = _dinv(degp_ref)
  g_ref[...] = jnp.dot(x_ref[...], w_ref[...],
                       preferred_element_type=jnp.float32) * dinv


def _tc2_body(qp_ref, g1_ref, degp_ref, b_ref, w_ref, g2_ref):
  dinv = _dinv(degp_ref)
  a = qp_ref[0] + qp_ref[1] + g1_ref[...]
  a = jnp.maximum(dinv * a + b_ref[...], 0.0)
  g2_ref[...] = jnp.dot(a, w_ref[...],
                        preferred_element_type=jnp.float32) * dinv


def _tc3_body(qp_ref, g2_ref, degp_ref, b_ref, out_ref):
  dinv = _dinv(degp_ref)
  out_ref[...] = dinv * (qp_ref[0] + qp_ref[1] + g2_ref[...]) + b_ref[...]


_row_spec = pl.BlockSpec((BR, D), lambda i: (i, 0))
_qp_spec = pl.BlockSpec((NC, BR, D), lambda i: (0, i, 0))
_degp_spec = pl.BlockSpec((NC, BR, D), lambda i: (0, i, 0))
_w_spec = pl.BlockSpec((D, D), lambda i: (0, 0))
_b_spec = pl.BlockSpec((1, D), lambda i: (0, 0))
_GRID = (N // BR,)

_tc1_call = pl.pallas_call(
    _tc1_body,
    grid=_GRID,
    in_specs=[_row_spec, _w_spec, _degp_spec],
    out_specs=_row_spec,
    out_shape=jax.ShapeDtypeStruct((N, D), jnp.float32),
)

_tc2_call = pl.pallas_call(
    _tc2_body,
    grid=_GRID,
    in_specs=[_qp_spec, _row_spec, _degp_spec, _b_spec, _w_spec],
    out_specs=_row_spec,
    out_shape=jax.ShapeDtypeStruct((N, D), jnp.float32),
)

_tc3_call = pl.pallas_call(
    _tc3_body,
    grid=_GRID,
    in_specs=[_qp_spec, _row_spec, _degp_spec, _b_spec],
    out_specs=_row_spec,
    out_shape=jax.ShapeDtypeStruct((N, D), jnp.float32),
)


@jax.jit
def kernel(x, edge_index, W1, b1, W2, b2):
  src_flat = edge_index[0]
  dst_flat = edge_index[1]
  onesD = jnp.ones((CHUNK, D), jnp.float32)
  zerosD = jnp.zeros((NPAD, D), jnp.float32)

  degp = _deg_call(dst_flat, onesD, zerosD)
  g1 = _tc1_call(x, W1, degp)
  q1 = _agg_call(g1, src_flat, dst_flat, zerosD)
  g2 = _tc2_call(q1, g1, degp, b1.reshape(1, D), W2)
  q2 = _agg_call(g2, src_flat, dst_flat, zerosD)
  return _tc3_call(q2, g2, degp, b2.reshape(1, D))
